# baseline (speedup 1.0000x reference)
import functools

import jax
import jax.numpy as jnp
from jax import lax
from jax.experimental import pallas as pl
from jax.experimental.pallas import tpu as pltpu
from jax.experimental.pallas import tpu_sc as plsc

VOCAB = 100000
EMB_DIM = 64
BATCH = 4096
HIST = 50

_NC, _NS = 2, 16
_NW = _NC * _NS
_B_PER_W = BATCH // _NW
_IDX_PER_W = _B_PER_W * HIST
_GROUPS = _B_PER_W // 16

_NCHUNK = 16
_CHUNK = VOCAB // _NCHUNK
_NBUF = 8


def _tc_matvec_body(emb_hbm, w_ref, p_ref, *scratch):
    bufs, sems = scratch[:_NBUF], scratch[_NBUF:]

    def start(i):
        pltpu.make_async_copy(
            emb_hbm.at[pl.ds(i * _CHUNK, _CHUNK), :],
            bufs[i % _NBUF],
            sems[i % _NBUF],
        ).start()

    for i in range(_NBUF):
        start(i)
    for i in range(_NCHUNK):
        pltpu.make_async_copy(
            emb_hbm.at[pl.ds(i * _CHUNK, _CHUNK), :],
            bufs[i % _NBUF],
            sems[i % _NBUF],
        ).wait()
        pt = lax.dot_general(
            w_ref[...], bufs[i % _NBUF][...],
            dimension_numbers=(((1,), (1,)), ((), ())),
            preferred_element_type=jnp.float32,
        )
        p_ref[pl.ds(i, 1), :] = pt
        if i + _NBUF < _NCHUNK:
            start(i + _NBUF)


_tc_matvec = pl.pallas_call(
    _tc_matvec_body,
    in_specs=[
        pl.BlockSpec(memory_space=pl.ANY),
        pl.BlockSpec((1, EMB_DIM), lambda: (0, 0)),
    ],
    out_specs=pl.BlockSpec((_NCHUNK, _CHUNK), lambda: (0, 0)),
    out_shape=jax.ShapeDtypeStruct((_NCHUNK, _CHUNK), jnp.float32),
    scratch_shapes=(
        [pltpu.VMEM((_CHUNK, EMB_DIM), jnp.float32)] * _NBUF
        + [pltpu.SemaphoreType.DMA] * _NBUF
    ),
)


def _sc_body(p_hbm, xf_hbm, mask_hbm, b_hbm, out_hbm, p_v, xf_v, mask_v, b_v, out_v):
    wid = lax.axis_index("s") * _NC + lax.axis_index("c")
    pltpu.sync_copy(p_hbm, p_v)
    pltpu.sync_copy(xf_hbm.at[pl.ds(wid * _IDX_PER_W, _IDX_PER_W)], xf_v)
    pltpu.sync_copy(mask_hbm.at[pl.ds(wid * _B_PER_W, _B_PER_W)], mask_v)
    pltpu.sync_copy(b_hbm, b_v)

    lane_off = lax.iota(jnp.int32, 16) * HIST

    for g in range(_GROUPS):
        def body(t, acc, g=g):
            offs = lane_off + (g * 16 * HIST + t)
            xi = plsc.load_gather(xf_v, [offs])
            rows = xi // _CHUNK
            cols = xi - rows * _CHUNK
            return acc + plsc.load_gather(p_v, [rows, cols])

        acc = lax.fori_loop(0, HIST, body, jnp.zeros((16,), jnp.float32))
        val = acc / mask_v[pl.ds(g * 16, 16)] + b_v[...]
        out_v[pl.ds(g * 16, 16)] = 1.0 / (1.0 + jnp.exp(-val))

    pltpu.sync_copy(out_v, out_hbm.at[pl.ds(wid * _B_PER_W, _B_PER_W)])


@functools.cache
def _sc_pool():
    return pl.kernel(
        _sc_body,
        out_type=jax.ShapeDtypeStruct((BATCH,), jnp.float32),
        mesh=plsc.VectorSubcoreMesh(
            core_axis_name="c", subcore_axis_name="s", num_cores=_NC, num_subcores=_NS
        ),
        compiler_params=pltpu.CompilerParams(needs_layout_passes=False),
        scratch_types=[
            pltpu.VMEM((_NCHUNK, _CHUNK), jnp.float32),
            pltpu.VMEM((_IDX_PER_W,), jnp.int32),
            pltpu.VMEM((_B_PER_W,), jnp.float32),
            pltpu.VMEM((16,), jnp.float32),
            pltpu.VMEM((_B_PER_W,), jnp.float32),
        ],
    )


def kernel(X, X_mask, emb, W, b):
    p = _tc_matvec(emb, W)
    xf = X.reshape(BATCH * HIST)
    mask = X_mask.reshape(BATCH)
    b16 = jnp.broadcast_to(b.astype(jnp.float32), (16,))
    return _sc_pool()(p, xf, mask, b16)
